# baseline (speedup 1.0000x reference)
import functools

import jax
import jax.numpy as jnp
from jax import lax
from jax.experimental import pallas as pl
from jax.experimental.pallas import tpu as pltpu
from jax.experimental.pallas import tpu_sc as plsc

NC = 2
NS = 16
NW = NC * NS
CH = 64
NBUF = 4
NSTAGE = 4
DEG_CH = 128
PAD_ROWS = 240


def _sc_mesh():
    return plsc.VectorSubcoreMesh(
        core_axis_name="c", subcore_axis_name="s", num_cores=NC, num_subcores=NS
    )


def _build_deg_kernel(n_pad, cpt):
    stripe = n_pad // NS

    @functools.partial(
        pl.kernel,
        out_type=jax.ShapeDtypeStruct((NC, n_pad), jnp.float32),
        mesh=_sc_mesh(),
        scratch_types=[
            pltpu.VMEM((cpt, DEG_CH), jnp.int32),
            pltpu.VMEM((DEG_CH,), jnp.float32),
            pltpu.VMEM_SHARED((n_pad,), jnp.float32),
        ],
    )
    def deg_kernel(dst_hbm, zeros_hbm, out_hbm, dst_v, ones_v, acc):
        cid = lax.axis_index("c")
        sid = lax.axis_index("s")
        wid = cid * NS + sid
        pltpu.sync_copy(dst_hbm.at[wid], dst_v)
        for k in range(DEG_CH // 16):
            ones_v[pl.ds(k * 16, 16)] = jnp.full((16,), 1.0, jnp.float32)
        pltpu.sync_copy(zeros_hbm, acc.at[pl.ds(sid * stripe, stripe)])
        plsc.subcore_barrier()

        @pl.loop(0, cpt)
        def _(j):
            pltpu.sync_copy(ones_v, acc.at[dst_v.at[j]], add=True)

        plsc.subcore_barrier()
        pltpu.sync_copy(
            acc.at[pl.ds(sid * stripe, stripe)],
            out_hbm.at[cid, pl.ds(sid * stripe, stripe)],
        )

    return deg_kernel


def _build_scatter_kernel(n_pad, cpt, d):
    stripe = n_pad // NS
    assert cpt % (NSTAGE * NBUF) == 0
    stage = cpt // NSTAGE

    @functools.partial(
        pl.kernel,
        out_type=jax.ShapeDtypeStruct((NC, n_pad, d), jnp.float32),
        mesh=_sc_mesh(),
        scratch_types=[
            pltpu.VMEM((stage, CH), jnp.int32),
            pltpu.VMEM((stage, CH), jnp.int32),
            [pltpu.VMEM((CH, d), jnp.float32) for _ in range(NBUF)],
            pltpu.VMEM_SHARED((n_pad, d), jnp.float32),
            [pltpu.SemaphoreType.DMA for _ in range(NBUF)],
            [pltpu.SemaphoreType.DMA for _ in range(NBUF)],
        ],
    )
    def scat_kernel(src_hbm, dst_hbm, h2_hbm, zeros_hbm, out_hbm,
                    src_v, dst_v, bufs, acc, gsems, ssems):
        cid = lax.axis_index("c")
        sid = lax.axis_index("s")
        wid = cid * NS + sid
        pltpu.sync_copy(zeros_hbm, acc.at[pl.ds(sid * stripe, stripe)])
        plsc.subcore_barrier()

        for st in range(NSTAGE):
            pltpu.sync_copy(src_hbm.at[wid, pl.ds(st * stage, stage)], src_v)
            pltpu.sync_copy(dst_hbm.at[wid, pl.ds(st * stage, stage)], dst_v)

            for k in range(NBUF):
                pltpu.async_copy(h2_hbm.at[src_v.at[k]], bufs[k], gsems[k])

            @pl.loop(0, stage, step=NBUF)
            def _(j):
                descs = []
                for k in range(NBUF):
                    pltpu.make_async_copy(
                        h2_hbm.at[src_v.at[j + k]], bufs[k], gsems[k]
                    ).wait()
                    descs.append(pltpu.async_copy(
                        bufs[k], acc.at[dst_v.at[j + k]], ssems[k], add=True
                    ))
                for k in range(NBUF):
                    descs[k].wait()

                    @pl.when(j + NBUF + k < stage)
                    def _():
                        pltpu.async_copy(
                            h2_hbm.at[src_v.at[j + NBUF + k]], bufs[k], gsems[k]
                        )

        plsc.subcore_barrier()
        pltpu.sync_copy(
            acc.at[pl.ds(sid * stripe, stripe)],
            out_hbm.at[cid, pl.ds(sid * stripe, stripe)],
        )

    return scat_kernel


def _matmul_scale(deg_t, x_pad, W, blk):
    n_pad, d_in = x_pad.shape
    d_out = W.shape[0]

    def body(dp_ref, x_ref, w_ref, h2_ref):
        dis = lax.rsqrt(dp_ref[:, 0:1] + dp_ref[:, 1:2] + 1.0)
        h2_ref[...] = lax.dot_general(
            x_ref[...] * dis, w_ref[...], (((1,), (1,)), ((), ())),
            preferred_element_type=jnp.float32,
            precision=lax.Precision.HIGHEST,
        )

    return pl.pallas_call(
        body,
        grid=(n_pad // blk,),
        in_specs=[
            pl.BlockSpec((blk, NC), lambda i: (i, 0)),
            pl.BlockSpec((blk, d_in), lambda i: (i, 0)),
            pl.BlockSpec((d_out, d_in), lambda i: (0, 0)),
        ],
        out_specs=pl.BlockSpec((blk, d_out), lambda i: (i, 0)),
        out_shape=jax.ShapeDtypeStruct((n_pad, d_out), jnp.float32),
    )(deg_t, x_pad, W)


def _epilogue(acc_partials, deg_t, h2, b2, blk):
    nc, n_pad, d = acc_partials.shape

    def body(ap_ref, dp_ref, h2_ref, b_ref, o_ref):
        dis = lax.rsqrt(dp_ref[:, 0:1] + dp_ref[:, 1:2] + 1.0)
        s = ap_ref[0] + ap_ref[1] + h2_ref[...]
        o_ref[...] = s * dis + b_ref[...]

    return pl.pallas_call(
        body,
        grid=(n_pad // blk,),
        in_specs=[
            pl.BlockSpec((nc, blk, d), lambda i: (0, i, 0)),
            pl.BlockSpec((blk, NC), lambda i: (i, 0)),
            pl.BlockSpec((blk, d), lambda i: (i, 0)),
            pl.BlockSpec((1, d), lambda i: (0, 0)),
        ],
        out_specs=pl.BlockSpec((blk, d), lambda i: (i, 0)),
        out_shape=jax.ShapeDtypeStruct((n_pad, d), jnp.float32),
    )(acc_partials, deg_t, h2, b2)


def kernel(x, edge_index, edge_attr, W, b):
    n, d_in = x.shape
    d = W.shape[0]
    e = edge_index.shape[1]

    align = NS * 8
    n_pad = ((n + PAD_ROWS + align - 1) // align) * align
    cpt = -(-e // (NW * CH))
    rnd = NSTAGE * NBUF
    cpt = ((cpt + rnd - 1) // rnd) * rnd
    assert (cpt * CH) % DEG_CH == 0
    e_pad = NW * cpt * CH
    stripe = n_pad // NS

    src = edge_index[0].astype(jnp.int32)
    dst = edge_index[1].astype(jnp.int32)
    npad_e = e_pad - e
    pad_i = jnp.arange(npad_e, dtype=jnp.int32)
    src_all = jnp.concatenate([src, pad_i % n]).reshape(NW, cpt, CH)
    dst_all = jnp.concatenate([dst, n + pad_i % (n_pad - n)]).reshape(NW, cpt, CH)
    dst_deg = dst_all.reshape(NW, (cpt * CH) // DEG_CH, DEG_CH)

    zeros1 = jnp.zeros((stripe,), jnp.float32)
    zeros2 = jnp.zeros((stripe, d), jnp.float32)
    x_pad = jnp.pad(x, ((0, n_pad - n), (0, 0)))

    deg_partials = _build_deg_kernel(n_pad, dst_deg.shape[1])(dst_deg, zeros1)
    deg_t = deg_partials.T
    h2 = _matmul_scale(deg_t, x_pad, W, blk=1280)
    acc_partials = _build_scatter_kernel(n_pad, cpt, d)(src_all, dst_all, h2, zeros2)
    out = _epilogue(acc_partials, deg_t, h2, b.reshape(1, d), blk=1280)
    return out[:n]


# speedup vs baseline: 1.0683x; 1.0683x over previous
import functools

import jax
import jax.numpy as jnp
import numpy as np
from jax import lax
from jax.experimental import pallas as pl
from jax.experimental.pallas import tpu as pltpu
from jax.experimental.pallas import tpu_sc as plsc

NC = 2
NS = 16
NW = NC * NS
CH = 128
PAD_ROWS = 240


def _sc_mesh():
    return plsc.VectorSubcoreMesh(
        core_axis_name="c", subcore_axis_name="s", num_cores=NC, num_subcores=NS
    )


def _build_deg_kernel(n_pad, cpt):
    stripe = n_pad // NS

    @functools.partial(
        pl.kernel,
        out_type=[
            jax.ShapeDtypeStruct((n_pad,), jnp.float32),
            jax.ShapeDtypeStruct((n_pad,), jnp.float32),
        ],
        mesh=_sc_mesh(),
        scratch_types=[
            pltpu.VMEM((cpt, CH), jnp.int32),
            pltpu.VMEM((CH,), jnp.float32),
            pltpu.VMEM_SHARED((n_pad,), jnp.float32),
        ],
    )
    def deg_kernel(dst_hbm, zeros_hbm, out0_hbm, out1_hbm, dst_v, ones_v, acc):
        cid = lax.axis_index("c")
        sid = lax.axis_index("s")
        wid = cid * NS + sid
        pltpu.sync_copy(dst_hbm.at[wid], dst_v)
        for k in range(CH // 16):
            ones_v[pl.ds(k * 16, 16)] = jnp.full((16,), 1.0, jnp.float32)
        pltpu.sync_copy(zeros_hbm, acc.at[pl.ds(sid * stripe, stripe)])
        plsc.subcore_barrier()

        @pl.loop(0, cpt)
        def _(j):
            pltpu.sync_copy(ones_v, acc.at[dst_v.at[j]], add=True)

        plsc.subcore_barrier()
        sl = pl.ds(sid * stripe, stripe)

        @pl.when(cid == 0)
        def _():
            pltpu.sync_copy(acc.at[sl], out0_hbm.at[sl])

        @pl.when(cid == 1)
        def _():
            pltpu.sync_copy(acc.at[sl], out1_hbm.at[sl])

    return deg_kernel


def _build_scatter_kernel(n_pad, cpt, d):
    stripe = n_pad // NS
    assert cpt % 4 == 0
    half = cpt // 2

    @functools.partial(
        pl.kernel,
        out_type=jax.ShapeDtypeStruct((NC, n_pad, d), jnp.float32),
        mesh=_sc_mesh(),
        scratch_types=[
            pltpu.VMEM((half, CH), jnp.int32),
            pltpu.VMEM((half, CH), jnp.int32),
            pltpu.VMEM((CH, d), jnp.float32),
            pltpu.VMEM((CH, d), jnp.float32),
            pltpu.VMEM_SHARED((n_pad, d), jnp.float32),
            pltpu.SemaphoreType.DMA,
            pltpu.SemaphoreType.DMA,
        ],
    )
    def scat_kernel(src_hbm, dst_hbm, h2_hbm, zeros_hbm, out_hbm,
                    src_v, dst_v, bufa, bufb, acc, sema, semb):
        cid = lax.axis_index("c")
        sid = lax.axis_index("s")
        wid = cid * NS + sid
        pltpu.sync_copy(zeros_hbm, acc.at[pl.ds(sid * stripe, stripe)])
        plsc.subcore_barrier()

        for hf in range(2):
            pltpu.sync_copy(src_hbm.at[wid, pl.ds(hf * half, half)], src_v)
            pltpu.sync_copy(dst_hbm.at[wid, pl.ds(hf * half, half)], dst_v)

            pltpu.async_copy(h2_hbm.at[src_v.at[0]], bufa, sema)

            @pl.loop(0, half, step=2)
            def _(j):
                pltpu.async_copy(h2_hbm.at[src_v.at[j + 1]], bufb, semb)
                pltpu.make_async_copy(h2_hbm.at[src_v.at[j]], bufa, sema).wait()
                pltpu.sync_copy(bufa, acc.at[dst_v.at[j]], add=True)

                @pl.when(j + 2 < half)
                def _():
                    pltpu.async_copy(h2_hbm.at[src_v.at[j + 2]], bufa, sema)

                pltpu.make_async_copy(h2_hbm.at[src_v.at[j + 1]], bufb, semb).wait()
                pltpu.sync_copy(bufb, acc.at[dst_v.at[j + 1]], add=True)

        plsc.subcore_barrier()
        pltpu.sync_copy(
            acc.at[pl.ds(sid * stripe, stripe)],
            out_hbm.at[cid, pl.ds(sid * stripe, stripe)],
        )

    return scat_kernel


def _matmul_scale(deg0, deg1, x, W, blk):
    n, d_in = x.shape
    d_out = W.shape[0]

    def body(d0_ref, d1_ref, x_ref, w_ref, h2_ref):
        dis = lax.rsqrt(d0_ref[...] + d1_ref[...] + 1.0)
        h2_ref[...] = lax.dot_general(
            x_ref[...] * dis, w_ref[...], (((1,), (1,)), ((), ())),
            preferred_element_type=jnp.float32,
        )

    return pl.pallas_call(
        body,
        grid=(n // blk,),
        in_specs=[
            pl.BlockSpec((blk, 1), lambda i: (i, 0)),
            pl.BlockSpec((blk, 1), lambda i: (i, 0)),
            pl.BlockSpec((blk, d_in), lambda i: (i, 0)),
            pl.BlockSpec((d_out, d_in), lambda i: (0, 0)),
        ],
        out_specs=pl.BlockSpec((blk, d_out), lambda i: (i, 0)),
        out_shape=jax.ShapeDtypeStruct((n, d_out), jnp.float32),
    )(deg0, deg1, x, W)


def _epilogue(acc_partials, deg0, deg1, h2, b2, blk):
    nc, n_pad, d = acc_partials.shape
    n = h2.shape[0]

    def body(ap_ref, d0_ref, d1_ref, h2_ref, b_ref, o_ref):
        dis = lax.rsqrt(d0_ref[...] + d1_ref[...] + 1.0)
        s = ap_ref[0] + ap_ref[1] + h2_ref[...]
        o_ref[...] = s * dis + b_ref[...]

    return pl.pallas_call(
        body,
        grid=(n // blk,),
        in_specs=[
            pl.BlockSpec((nc, blk, d), lambda i: (0, i, 0)),
            pl.BlockSpec((blk, 1), lambda i: (i, 0)),
            pl.BlockSpec((blk, 1), lambda i: (i, 0)),
            pl.BlockSpec((blk, d), lambda i: (i, 0)),
            pl.BlockSpec((1, d), lambda i: (0, 0)),
        ],
        out_specs=pl.BlockSpec((blk, d), lambda i: (i, 0)),
        out_shape=jax.ShapeDtypeStruct((n, d), jnp.float32),
    )(acc_partials, deg0, deg1, h2, b2)


def kernel(x, edge_index, edge_attr, W, b):
    n, d_in = x.shape
    d = W.shape[0]
    e = edge_index.shape[1]

    align = NS * 8
    n_pad = ((n + PAD_ROWS + align - 1) // align) * align
    cpt = -(-e // (NW * CH))
    cpt = ((cpt + 3) // 4) * 4
    e_pad = NW * cpt * CH
    stripe = n_pad // NS

    src = edge_index[0].astype(jnp.int32)
    dst = edge_index[1].astype(jnp.int32)
    npad_e = e_pad - e
    pad_i = np.arange(npad_e, dtype=np.int32)
    pad_src = jnp.asarray(pad_i % n)
    pad_dst = jnp.asarray(n + pad_i % (n_pad - n))
    src_all = jnp.concatenate([src, pad_src]).reshape(NW, cpt, CH)
    dst_all = jnp.concatenate([dst, pad_dst]).reshape(NW, cpt, CH)

    zeros1 = jnp.zeros((stripe,), jnp.float32)
    zeros2 = jnp.zeros((stripe, d), jnp.float32)

    deg0, deg1 = _build_deg_kernel(n_pad, cpt)(dst_all, zeros1)
    deg0 = deg0.reshape(n_pad, 1)
    deg1 = deg1.reshape(n_pad, 1)
    h2 = _matmul_scale(deg0, deg1, x, W, blk=1000)
    acc_partials = _build_scatter_kernel(n_pad, cpt, d)(src_all, dst_all, h2, zeros2)
    return _epilogue(acc_partials, deg0, deg1, h2, b.reshape(1, d), blk=1000)


# speedup vs baseline: 1.1164x; 1.0451x over previous
import functools

import jax
import jax.numpy as jnp
import numpy as np
from jax import lax
from jax.experimental import pallas as pl
from jax.experimental.pallas import tpu as pltpu
from jax.experimental.pallas import tpu_sc as plsc

NC = 2
NS = 16
NW = NC * NS
CH = 128
PAD_ROWS = 112


def _sc_mesh():
    return plsc.VectorSubcoreMesh(
        core_axis_name="c", subcore_axis_name="s", num_cores=NC, num_subcores=NS
    )


def _build_deg_kernel(n_pad, cpt):
    stripe = n_pad // NS

    @functools.partial(
        pl.kernel,
        out_type=[
            jax.ShapeDtypeStruct((n_pad,), jnp.float32),
            jax.ShapeDtypeStruct((n_pad,), jnp.float32),
        ],
        mesh=_sc_mesh(),
        scratch_types=[
            pltpu.VMEM((cpt, CH), jnp.int32),
            pltpu.VMEM((CH,), jnp.float32),
            pltpu.VMEM_SHARED((n_pad,), jnp.float32),
        ],
    )
    def deg_kernel(ei_hbm, zeros_hbm, out0_hbm, out1_hbm, dst_v, ones_v, acc):
        cid = lax.axis_index("c")
        sid = lax.axis_index("s")
        wid = cid * NS + sid
        pltpu.sync_copy(ei_hbm.at[1, wid], dst_v)
        for k in range(CH // 16):
            ones_v[pl.ds(k * 16, 16)] = jnp.full((16,), 1.0, jnp.float32)
        pltpu.sync_copy(zeros_hbm, acc.at[pl.ds(sid * stripe, stripe)])
        plsc.subcore_barrier()

        @pl.loop(0, cpt)
        def _(j):
            pltpu.sync_copy(ones_v, acc.at[dst_v.at[j]], add=True)

        plsc.subcore_barrier()
        sl = pl.ds(sid * stripe, stripe)

        @pl.when(cid == 0)
        def _():
            pltpu.sync_copy(acc.at[sl], out0_hbm.at[sl])

        @pl.when(cid == 1)
        def _():
            pltpu.sync_copy(acc.at[sl], out1_hbm.at[sl])

    return deg_kernel


def _build_scatter_kernel(n_pad, cpt, d):
    stripe = n_pad // NS
    assert cpt % 4 == 0
    half = cpt // 2

    @functools.partial(
        pl.kernel,
        out_type=jax.ShapeDtypeStruct((NC, n_pad, d), jnp.float32),
        mesh=_sc_mesh(),
        scratch_types=[
            pltpu.VMEM((half, CH), jnp.int32),
            pltpu.VMEM((half, CH), jnp.int32),
            pltpu.VMEM((CH, d), jnp.float32),
            pltpu.VMEM((CH, d), jnp.float32),
            pltpu.VMEM_SHARED((n_pad, d), jnp.float32),
            pltpu.SemaphoreType.DMA,
            pltpu.SemaphoreType.DMA,
        ],
    )
    def scat_kernel(ei_hbm, h2_hbm, zeros_hbm, out_hbm,
                    src_v, dst_v, bufa, bufb, acc, sema, semb):
        cid = lax.axis_index("c")
        sid = lax.axis_index("s")
        wid = cid * NS + sid
        pltpu.sync_copy(zeros_hbm, acc.at[pl.ds(sid * stripe, stripe)])
        plsc.subcore_barrier()

        for hf in range(2):
            pltpu.sync_copy(ei_hbm.at[0, wid, pl.ds(hf * half, half)], src_v)
            pltpu.sync_copy(ei_hbm.at[1, wid, pl.ds(hf * half, half)], dst_v)

            pltpu.async_copy(h2_hbm.at[src_v.at[0]], bufa, sema)

            @pl.loop(0, half, step=2)
            def _(j):
                pltpu.async_copy(h2_hbm.at[src_v.at[j + 1]], bufb, semb)
                pltpu.make_async_copy(h2_hbm.at[src_v.at[j]], bufa, sema).wait()
                pltpu.sync_copy(bufa, acc.at[dst_v.at[j]], add=True)

                @pl.when(j + 2 < half)
                def _():
                    pltpu.async_copy(h2_hbm.at[src_v.at[j + 2]], bufa, sema)

                pltpu.make_async_copy(h2_hbm.at[src_v.at[j + 1]], bufb, semb).wait()
                pltpu.sync_copy(bufb, acc.at[dst_v.at[j + 1]], add=True)

        plsc.subcore_barrier()
        pltpu.sync_copy(
            acc.at[pl.ds(sid * stripe, stripe)],
            out_hbm.at[cid, pl.ds(sid * stripe, stripe)],
        )

    return scat_kernel


def _matmul_scale(deg0, deg1, x, W, blk):
    n, d_in = x.shape
    d_out = W.shape[0]

    def body(d0_ref, d1_ref, x_ref, w_ref, h2_ref):
        dis = lax.rsqrt(d0_ref[...] + d1_ref[...] + 1.0)
        h2_ref[...] = lax.dot_general(
            x_ref[...] * dis, w_ref[...], (((1,), (1,)), ((), ())),
            preferred_element_type=jnp.float32,
        )

    return pl.pallas_call(
        body,
        grid=(n // blk,),
        in_specs=[
            pl.BlockSpec((blk, 1), lambda i: (i, 0)),
            pl.BlockSpec((blk, 1), lambda i: (i, 0)),
            pl.BlockSpec((blk, d_in), lambda i: (i, 0)),
            pl.BlockSpec((d_out, d_in), lambda i: (0, 0)),
        ],
        out_specs=pl.BlockSpec((blk, d_out), lambda i: (i, 0)),
        out_shape=jax.ShapeDtypeStruct((n, d_out), jnp.float32),
    )(deg0, deg1, x, W)


def _epilogue(acc_partials, deg0, deg1, h2, b2, blk):
    nc, n_pad, d = acc_partials.shape
    n = h2.shape[0]

    def body(ap_ref, d0_ref, d1_ref, h2_ref, b_ref, o_ref):
        dis = lax.rsqrt(d0_ref[...] + d1_ref[...] + 1.0)
        s = ap_ref[0] + ap_ref[1] + h2_ref[...]
        o_ref[...] = s * dis + b_ref[...]

    return pl.pallas_call(
        body,
        grid=(n // blk,),
        in_specs=[
            pl.BlockSpec((nc, blk, d), lambda i: (0, i, 0)),
            pl.BlockSpec((blk, 1), lambda i: (i, 0)),
            pl.BlockSpec((blk, 1), lambda i: (i, 0)),
            pl.BlockSpec((blk, d), lambda i: (i, 0)),
            pl.BlockSpec((1, d), lambda i: (0, 0)),
        ],
        out_specs=pl.BlockSpec((blk, d), lambda i: (i, 0)),
        out_shape=jax.ShapeDtypeStruct((n, d), jnp.float32),
    )(acc_partials, deg0, deg1, h2, b2)


def kernel(x, edge_index, edge_attr, W, b):
    n, d_in = x.shape
    d = W.shape[0]
    e = edge_index.shape[1]

    align = NS * 128
    n_pad = ((n + PAD_ROWS + align - 1) // align) * align
    cpt = -(-e // (NW * CH))
    cpt = ((cpt + 3) // 4) * 4
    e_pad = NW * cpt * CH
    stripe = n_pad // NS

    ei = edge_index.astype(jnp.int32)
    npad_e = e_pad - e
    pad_i = np.arange(npad_e, dtype=np.int32)
    pad2 = jnp.asarray(np.stack([pad_i % n, n + pad_i % (n_pad - n)]))
    ei_all = jnp.concatenate([ei, pad2], axis=1).reshape(2, NW, cpt, CH)

    zeros1 = jnp.zeros((stripe,), jnp.float32)
    zeros2 = jnp.zeros((stripe, d), jnp.float32)

    deg0, deg1 = _build_deg_kernel(n_pad, cpt)(ei_all, zeros1)
    deg0 = deg0.reshape(n_pad, 1)
    deg1 = deg1.reshape(n_pad, 1)
    h2 = _matmul_scale(deg0, deg1, x, W, blk=1000)
    acc_partials = _build_scatter_kernel(n_pad, cpt, d)(ei_all, h2, zeros2)
    return _epilogue(acc_partials, deg0, deg1, h2, b.reshape(1, d), blk=1000)


# speedup vs baseline: 1.1905x; 1.0663x over previous
import functools

import jax
import jax.numpy as jnp
import numpy as np
from jax import lax
from jax.experimental import pallas as pl
from jax.experimental.pallas import tpu as pltpu
from jax.experimental.pallas import tpu_sc as plsc

NC = 2
NS = 16
NW = NC * NS
CH = 128
PAD_ROWS = 112


def _sc_mesh():
    return plsc.VectorSubcoreMesh(
        core_axis_name="c", subcore_axis_name="s", num_cores=NC, num_subcores=NS
    )


def _build_deg_kernel(n_pad, cpt):
    stripe = n_pad // NS

    @functools.partial(
        pl.kernel,
        out_type=[
            jax.ShapeDtypeStruct((n_pad,), jnp.float32),
            jax.ShapeDtypeStruct((n_pad,), jnp.float32),
        ],
        mesh=_sc_mesh(),
        scratch_types=[
            pltpu.VMEM((cpt, CH), jnp.int32),
            pltpu.VMEM((CH,), jnp.float32),
            pltpu.VMEM_SHARED((n_pad,), jnp.float32),
        ],
    )
    def deg_kernel(ei_hbm, zeros_hbm, out0_hbm, out1_hbm, dst_v, ones_v, acc):
        cid = lax.axis_index("c")
        sid = lax.axis_index("s")
        wid = cid * NS + sid
        pltpu.sync_copy(ei_hbm.at[1, wid], dst_v)
        for k in range(CH // 16):
            ones_v[pl.ds(k * 16, 16)] = jnp.full((16,), 1.0, jnp.float32)
        pltpu.sync_copy(zeros_hbm, acc.at[pl.ds(sid * stripe, stripe)])
        plsc.subcore_barrier()

        @pl.loop(0, cpt)
        def _(j):
            pltpu.sync_copy(ones_v, acc.at[dst_v.at[j]], add=True)

        plsc.subcore_barrier()
        sl = pl.ds(sid * stripe, stripe)

        @pl.when(cid == 0)
        def _():
            pltpu.sync_copy(acc.at[sl], out0_hbm.at[sl])

        @pl.when(cid == 1)
        def _():
            pltpu.sync_copy(acc.at[sl], out1_hbm.at[sl])

    return deg_kernel


def _build_scatter_kernel(n_pad, cpt, d):
    stripe = n_pad // NS
    assert cpt % 4 == 0
    half = cpt // 2

    @functools.partial(
        pl.kernel,
        out_type=jax.ShapeDtypeStruct((NC, n_pad, d), jnp.float32),
        mesh=_sc_mesh(),
        scratch_types=[
            pltpu.VMEM((half, CH), jnp.int32),
            pltpu.VMEM((half, CH), jnp.int32),
            pltpu.VMEM((CH, d), jnp.float32),
            pltpu.VMEM((CH, d), jnp.float32),
            pltpu.VMEM_SHARED((n_pad, d), jnp.float32),
            pltpu.SemaphoreType.DMA,
            pltpu.SemaphoreType.DMA,
        ],
    )
    def scat_kernel(ei_hbm, h2_hbm, zeros_hbm, out_hbm,
                    src_v, dst_v, bufa, bufb, acc, sema, semb):
        cid = lax.axis_index("c")
        sid = lax.axis_index("s")
        wid = cid * NS + sid
        pltpu.sync_copy(zeros_hbm, acc.at[pl.ds(sid * stripe, stripe)])
        plsc.subcore_barrier()

        for hf in range(2):
            pltpu.sync_copy(ei_hbm.at[0, wid, pl.ds(hf * half, half)], src_v)
            pltpu.sync_copy(ei_hbm.at[1, wid, pl.ds(hf * half, half)], dst_v)

            pltpu.async_copy(h2_hbm.at[src_v.at[0]], bufa, sema)

            @pl.loop(0, half, step=2)
            def _(j):
                pltpu.async_copy(h2_hbm.at[src_v.at[j + 1]], bufb, semb)
                pltpu.make_async_copy(h2_hbm.at[src_v.at[j]], bufa, sema).wait()
                pltpu.sync_copy(bufa, acc.at[dst_v.at[j]], add=True)

                @pl.when(j + 2 < half)
                def _():
                    pltpu.async_copy(h2_hbm.at[src_v.at[j + 2]], bufa, sema)

                pltpu.make_async_copy(h2_hbm.at[src_v.at[j + 1]], bufb, semb).wait()
                pltpu.sync_copy(bufb, acc.at[dst_v.at[j + 1]], add=True)

        plsc.subcore_barrier()
        pltpu.sync_copy(
            acc.at[pl.ds(sid * stripe, stripe)],
            out_hbm.at[cid, pl.ds(sid * stripe, stripe)],
        )

    return scat_kernel


def _dis_col(d0, d1, blk):
    r = d0.shape[0]
    dis = lax.rsqrt(d0 + d1 + 1.0)
    rep = jnp.broadcast_to(dis[:, None, :], (r, 128, 128)).reshape(blk, 128)
    row = lax.broadcasted_iota(jnp.int32, (blk, 128), 0)
    lane = lax.broadcasted_iota(jnp.int32, (blk, 128), 1)
    sel = jnp.where(lane == row % 128, rep, 0.0)
    return jnp.sum(sel, axis=1, keepdims=True)


def _matmul_scale(deg0, deg1, x, W, blk):
    n, d_in = x.shape
    d_out = W.shape[0]
    r = blk // 128

    def body(d0_ref, d1_ref, x_ref, w_ref, h2_ref):
        dis = _dis_col(d0_ref[...], d1_ref[...], blk)
        h2_ref[...] = lax.dot_general(
            x_ref[...] * dis, w_ref[...],
            (((1,), (1,)), ((), ())),
            preferred_element_type=jnp.float32,
        )

    grid = (deg0.shape[0] // r,)
    return pl.pallas_call(
        body,
        grid=grid,
        in_specs=[
            pl.BlockSpec((r, 128), lambda i: (i, 0)),
            pl.BlockSpec((r, 128), lambda i: (i, 0)),
            pl.BlockSpec((blk, d_in), lambda i: (i, 0)),
            pl.BlockSpec((d_out, d_in), lambda i: (0, 0)),
        ],
        out_specs=pl.BlockSpec((blk, d_out), lambda i: (i, 0)),
        out_shape=jax.ShapeDtypeStruct((n, d_out), jnp.float32),
    )(deg0, deg1, x, W)


def _epilogue(acc_partials, deg0, deg1, h2, b2, blk):
    nc, n_pad, d = acc_partials.shape
    n = h2.shape[0]
    r = blk // 128

    def body(ap_ref, d0_ref, d1_ref, h2_ref, b_ref, o_ref):
        dis = _dis_col(d0_ref[...], d1_ref[...], blk)
        s = ap_ref[0] + ap_ref[1] + h2_ref[...]
        o_ref[...] = s * dis + b_ref[...]

    return pl.pallas_call(
        body,
        grid=(n_pad // blk,),
        in_specs=[
            pl.BlockSpec((nc, blk, d), lambda i: (0, i, 0)),
            pl.BlockSpec((r, 128), lambda i: (i, 0)),
            pl.BlockSpec((r, 128), lambda i: (i, 0)),
            pl.BlockSpec((blk, d), lambda i: (i, 0)),
            pl.BlockSpec((1, d), lambda i: (0, 0)),
        ],
        out_specs=pl.BlockSpec((blk, d), lambda i: (i, 0)),
        out_shape=jax.ShapeDtypeStruct((n, d), jnp.float32),
    )(acc_partials, deg0, deg1, h2, b2)


def kernel(x, edge_index, edge_attr, W, b):
    n, d_in = x.shape
    d = W.shape[0]
    e = edge_index.shape[1]

    align = NS * 128
    n_pad = ((n + PAD_ROWS + align - 1) // align) * align
    cpt = -(-e // (NW * CH))
    cpt = ((cpt + 3) // 4) * 4
    e_pad = NW * cpt * CH
    stripe = n_pad // NS

    ei = edge_index.astype(jnp.int32)
    npad_e = e_pad - e
    pad_i = np.arange(npad_e, dtype=np.int32)
    pad2 = jnp.asarray(np.stack([pad_i % n, n + pad_i % (n_pad - n)]))
    ei_all = jnp.concatenate([ei, pad2], axis=1).reshape(2, NW, cpt, CH)

    zeros1 = jnp.zeros((stripe,), jnp.float32)
    zeros2 = jnp.zeros((stripe, d), jnp.float32)

    deg0, deg1 = _build_deg_kernel(n_pad, cpt)(ei_all, zeros1)
    deg0 = deg0.reshape(n_pad // 128, 128)
    deg1 = deg1.reshape(n_pad // 128, 128)
    h2 = _matmul_scale(deg0, deg1, x, W, blk=1024)
    acc_partials = _build_scatter_kernel(n_pad, cpt, d)(ei_all, h2, zeros2)
    return _epilogue(acc_partials, deg0, deg1, h2, b.reshape(1, d), blk=1024)


# speedup vs baseline: 1.1926x; 1.0018x over previous
import functools

import jax
import jax.numpy as jnp
import numpy as np
from jax import lax
from jax.experimental import pallas as pl
from jax.experimental.pallas import tpu as pltpu
from jax.experimental.pallas import tpu_sc as plsc

NC = 2
NS = 16
NW = NC * NS
CH = 128
PAD_ROWS = 112


def _sc_mesh():
    return plsc.VectorSubcoreMesh(
        core_axis_name="c", subcore_axis_name="s", num_cores=NC, num_subcores=NS
    )


def _build_deg_kernel(n_pad, cpt):
    stripe = n_pad // NS

    @functools.partial(
        pl.kernel,
        out_type=[
            jax.ShapeDtypeStruct((n_pad,), jnp.float32),
            jax.ShapeDtypeStruct((n_pad,), jnp.float32),
        ],
        mesh=_sc_mesh(),
        scratch_types=[
            pltpu.VMEM((cpt, CH), jnp.int32),
            pltpu.VMEM((CH,), jnp.float32),
            pltpu.VMEM_SHARED((n_pad,), jnp.float32),
            pltpu.SemaphoreType.DMA,
        ],
    )
    def deg_kernel(ei_hbm, zeros_hbm, out0_hbm, out1_hbm, dst_v, ones_v, acc, sem):
        cid = lax.axis_index("c")
        sid = lax.axis_index("s")
        wid = cid * NS + sid
        pltpu.sync_copy(ei_hbm.at[1, wid], dst_v)
        for k in range(CH // 16):
            ones_v[pl.ds(k * 16, 16)] = jnp.full((16,), 1.0, jnp.float32)
        pltpu.sync_copy(zeros_hbm, acc.at[pl.ds(sid * stripe, stripe)])
        plsc.subcore_barrier()

        @pl.loop(0, cpt)
        def _(j):
            pltpu.async_copy(ones_v, acc.at[dst_v.at[j]], sem, add=True)

        @pl.loop(0, cpt)
        def _(j):
            pltpu.make_async_copy(ones_v, acc.at[dst_v.at[j]], sem).wait()

        plsc.subcore_barrier()
        sl = pl.ds(sid * stripe, stripe)

        @pl.when(cid == 0)
        def _():
            pltpu.sync_copy(acc.at[sl], out0_hbm.at[sl])

        @pl.when(cid == 1)
        def _():
            pltpu.sync_copy(acc.at[sl], out1_hbm.at[sl])

    return deg_kernel


def _build_scatter_kernel(n_pad, cpt, d):
    stripe = n_pad // NS
    assert cpt % 4 == 0
    half = cpt // 2

    @functools.partial(
        pl.kernel,
        out_type=jax.ShapeDtypeStruct((NC, n_pad, d), jnp.float32),
        mesh=_sc_mesh(),
        scratch_types=[
            pltpu.VMEM((half, CH), jnp.int32),
            pltpu.VMEM((half, CH), jnp.int32),
            pltpu.VMEM((CH, d), jnp.float32),
            pltpu.VMEM((CH, d), jnp.float32),
            pltpu.VMEM_SHARED((n_pad, d), jnp.float32),
            pltpu.SemaphoreType.DMA,
            pltpu.SemaphoreType.DMA,
        ],
    )
    def scat_kernel(ei_hbm, h2_hbm, zeros_hbm, out_hbm,
                    src_v, dst_v, bufa, bufb, acc, sema, semb):
        cid = lax.axis_index("c")
        sid = lax.axis_index("s")
        wid = cid * NS + sid
        pltpu.sync_copy(zeros_hbm, acc.at[pl.ds(sid * stripe, stripe)])
        plsc.subcore_barrier()

        for hf in range(2):
            pltpu.sync_copy(ei_hbm.at[0, wid, pl.ds(hf * half, half)], src_v)
            pltpu.sync_copy(ei_hbm.at[1, wid, pl.ds(hf * half, half)], dst_v)

            pltpu.async_copy(h2_hbm.at[src_v.at[0]], bufa, sema)

            @pl.loop(0, half, step=2)
            def _(j):
                pltpu.async_copy(h2_hbm.at[src_v.at[j + 1]], bufb, semb)
                pltpu.make_async_copy(h2_hbm.at[src_v.at[j]], bufa, sema).wait()
                pltpu.sync_copy(bufa, acc.at[dst_v.at[j]], add=True)

                @pl.when(j + 2 < half)
                def _():
                    pltpu.async_copy(h2_hbm.at[src_v.at[j + 2]], bufa, sema)

                pltpu.make_async_copy(h2_hbm.at[src_v.at[j + 1]], bufb, semb).wait()
                pltpu.sync_copy(bufb, acc.at[dst_v.at[j + 1]], add=True)

        plsc.subcore_barrier()
        pltpu.sync_copy(
            acc.at[pl.ds(sid * stripe, stripe)],
            out_hbm.at[cid, pl.ds(sid * stripe, stripe)],
        )

    return scat_kernel


def _dis_col(d0, d1, blk):
    r = d0.shape[0]
    dis = lax.rsqrt(d0 + d1 + 1.0)
    rep = jnp.broadcast_to(dis[:, None, :], (r, 128, 128)).reshape(blk, 128)
    row = lax.broadcasted_iota(jnp.int32, (blk, 128), 0)
    lane = lax.broadcasted_iota(jnp.int32, (blk, 128), 1)
    sel = jnp.where(lane == row % 128, rep, 0.0)
    ones_col = jnp.full((128, 1), 1.0, jnp.float32)
    return lax.dot_general(sel, ones_col, (((1,), (0,)), ((), ())),
                           preferred_element_type=jnp.float32,
                           precision=lax.Precision.HIGHEST)


def _matmul_scale(deg0, deg1, x, W, blk):
    n, d_in = x.shape
    d_out = W.shape[0]
    r = blk // 128

    def body(d0_ref, d1_ref, x_ref, w_ref, h2_ref):
        dis = _dis_col(d0_ref[...], d1_ref[...], blk)
        h2_ref[...] = lax.dot_general(
            x_ref[...] * dis, w_ref[...],
            (((1,), (1,)), ((), ())),
            preferred_element_type=jnp.float32,
        )

    grid = (deg0.shape[0] // r,)
    return pl.pallas_call(
        body,
        grid=grid,
        in_specs=[
            pl.BlockSpec((r, 128), lambda i: (i, 0)),
            pl.BlockSpec((r, 128), lambda i: (i, 0)),
            pl.BlockSpec((blk, d_in), lambda i: (i, 0)),
            pl.BlockSpec((d_out, d_in), lambda i: (0, 0)),
        ],
        out_specs=pl.BlockSpec((blk, d_out), lambda i: (i, 0)),
        out_shape=jax.ShapeDtypeStruct((n, d_out), jnp.float32),
    )(deg0, deg1, x, W)


def _epilogue(acc_partials, deg0, deg1, h2, b2, blk):
    nc, n_pad, d = acc_partials.shape
    n = h2.shape[0]
    r = blk // 128

    def body(ap_ref, d0_ref, d1_ref, h2_ref, b_ref, o_ref):
        dis = _dis_col(d0_ref[...], d1_ref[...], blk)
        s = ap_ref[0] + ap_ref[1] + h2_ref[...]
        o_ref[...] = s * dis + b_ref[...]

    return pl.pallas_call(
        body,
        grid=(n_pad // blk,),
        in_specs=[
            pl.BlockSpec((nc, blk, d), lambda i: (0, i, 0)),
            pl.BlockSpec((r, 128), lambda i: (i, 0)),
            pl.BlockSpec((r, 128), lambda i: (i, 0)),
            pl.BlockSpec((blk, d), lambda i: (i, 0)),
            pl.BlockSpec((1, d), lambda i: (0, 0)),
        ],
        out_specs=pl.BlockSpec((blk, d), lambda i: (i, 0)),
        out_shape=jax.ShapeDtypeStruct((n, d), jnp.float32),
    )(acc_partials, deg0, deg1, h2, b2)


def kernel(x, edge_index, edge_attr, W, b):
    n, d_in = x.shape
    d = W.shape[0]
    e = edge_index.shape[1]

    align = NS * 128
    n_pad = ((n + PAD_ROWS + align - 1) // align) * align
    cpt = -(-e // (NW * CH))
    cpt = ((cpt + 3) // 4) * 4
    e_pad = NW * cpt * CH
    stripe = n_pad // NS

    ei = edge_index.astype(jnp.int32)
    npad_e = e_pad - e
    pad_i = np.arange(npad_e, dtype=np.int32)
    pad2 = jnp.asarray(np.stack([pad_i % n, n + pad_i % (n_pad - n)]))
    ei_all = jnp.concatenate([ei, pad2], axis=1).reshape(2, NW, cpt, CH)

    zeros1 = jnp.zeros((stripe,), jnp.float32)
    zeros2 = jnp.zeros((stripe, d), jnp.float32)

    deg0, deg1 = _build_deg_kernel(n_pad, cpt)(ei_all, zeros1)
    deg0 = deg0.reshape(n_pad // 128, 128)
    deg1 = deg1.reshape(n_pad // 128, 128)
    h2 = _matmul_scale(deg0, deg1, x, W, blk=1024)
    acc_partials = _build_scatter_kernel(n_pad, cpt, d)(ei_all, h2, zeros2)
    return _epilogue(acc_partials, deg0, deg1, h2, b.reshape(1, d), blk=1024)


# speedup vs baseline: 1.2619x; 1.0581x over previous
import functools

import jax
import jax.numpy as jnp
import numpy as np
from jax import lax
from jax.experimental import pallas as pl
from jax.experimental.pallas import tpu as pltpu
from jax.experimental.pallas import tpu_sc as plsc

NC = 2
NS = 16
NW = NC * NS
CH = 128
PAD_ROWS = 112


def _sc_mesh():
    return plsc.VectorSubcoreMesh(
        core_axis_name="c", subcore_axis_name="s", num_cores=NC, num_subcores=NS
    )


def _build_deg_kernel(n_pad, cpt):
    stripe = n_pad // NS

    @functools.partial(
        pl.kernel,
        out_type=[
            jax.ShapeDtypeStruct((n_pad,), jnp.float32),
            jax.ShapeDtypeStruct((n_pad,), jnp.float32),
        ],
        mesh=_sc_mesh(),
        scratch_types=[
            pltpu.VMEM((cpt, CH), jnp.int32),
            pltpu.VMEM((CH,), jnp.float32),
            pltpu.VMEM_SHARED((n_pad,), jnp.float32),
            pltpu.SemaphoreType.DMA,
        ],
    )
    def deg_kernel(ei_hbm, zeros_hbm, out0_hbm, out1_hbm, dst_v, ones_v, acc, sem):
        cid = lax.axis_index("c")
        sid = lax.axis_index("s")
        wid = cid * NS + sid
        pltpu.sync_copy(ei_hbm.at[1, wid], dst_v)
        for k in range(CH // 16):
            ones_v[pl.ds(k * 16, 16)] = jnp.full((16,), 1.0, jnp.float32)
        pltpu.sync_copy(zeros_hbm, acc.at[pl.ds(sid * stripe, stripe)])
        plsc.subcore_barrier()

        @pl.loop(0, cpt)
        def _(j):
            pltpu.async_copy(ones_v, acc.at[dst_v.at[j]], sem, add=True)

        @pl.loop(0, cpt)
        def _(j):
            pltpu.make_async_copy(ones_v, acc.at[dst_v.at[j]], sem).wait()

        plsc.subcore_barrier()
        sl = pl.ds(sid * stripe, stripe)

        @pl.when(cid == 0)
        def _():
            pltpu.sync_copy(acc.at[sl], out0_hbm.at[sl])

        @pl.when(cid == 1)
        def _():
            pltpu.sync_copy(acc.at[sl], out1_hbm.at[sl])

    return deg_kernel


def _build_scatter_kernel(n_pad, cpt, d):
    stripe = n_pad // NS
    assert cpt % 4 == 0
    half = cpt // 2

    @functools.partial(
        pl.kernel,
        out_type=jax.ShapeDtypeStruct((NC, n_pad, d), jnp.float32),
        mesh=_sc_mesh(),
        scratch_types=[
            pltpu.VMEM((half, CH), jnp.int32),
            pltpu.VMEM((half, CH), jnp.int32),
            pltpu.VMEM((CH, d), jnp.float32),
            pltpu.VMEM((CH, d), jnp.float32),
            pltpu.VMEM_SHARED((n_pad, d), jnp.float32),
            pltpu.SemaphoreType.DMA,
            pltpu.SemaphoreType.DMA,
        ],
    )
    def scat_kernel(ei_hbm, h2_hbm, zeros_hbm, out_hbm,
                    src_v, dst_v, bufa, bufb, acc, sema, semb):
        cid = lax.axis_index("c")
        sid = lax.axis_index("s")
        wid = cid * NS + sid
        pltpu.sync_copy(zeros_hbm, acc.at[pl.ds(sid * stripe, stripe)])
        plsc.subcore_barrier()

        for hf in range(2):
            pltpu.sync_copy(ei_hbm.at[0, wid, pl.ds(hf * half, half)], src_v)
            pltpu.sync_copy(ei_hbm.at[1, wid, pl.ds(hf * half, half)], dst_v)

            pltpu.async_copy(h2_hbm.at[src_v.at[0]], bufa, sema)

            @pl.loop(0, half, step=2)
            def _(j):
                pltpu.async_copy(h2_hbm.at[src_v.at[j + 1]], bufb, semb)
                pltpu.make_async_copy(h2_hbm.at[src_v.at[j]], bufa, sema).wait()
                pltpu.sync_copy(bufa, acc.at[dst_v.at[j]], add=True)

                @pl.when(j + 2 < half)
                def _():
                    pltpu.async_copy(h2_hbm.at[src_v.at[j + 2]], bufa, sema)

                pltpu.make_async_copy(h2_hbm.at[src_v.at[j + 1]], bufb, semb).wait()
                pltpu.sync_copy(bufb, acc.at[dst_v.at[j + 1]], add=True)

        plsc.subcore_barrier()
        pltpu.sync_copy(
            acc.at[pl.ds(sid * stripe, stripe)],
            out_hbm.at[cid, pl.ds(sid * stripe, stripe)],
        )

    return scat_kernel


def _dis_col(d0, d1, blk):
    r = d0.shape[0]
    dis = lax.rsqrt(d0 + d1 + 1.0)
    rep = jnp.broadcast_to(dis[:, None, :], (r, 128, 128)).reshape(blk, 128)
    row = lax.broadcasted_iota(jnp.int32, (blk, 128), 0)
    lane = lax.broadcasted_iota(jnp.int32, (blk, 128), 1)
    sel = jnp.where(lane == row % 128, rep, 0.0)
    return jnp.sum(sel, axis=1, keepdims=True)


def _matmul_scale(deg0, deg1, x, W, blk):
    n, d_in = x.shape
    d_out = W.shape[0]
    r = blk // 128

    def body(d0_ref, d1_ref, x_ref, w_ref, h2_ref):
        dis = _dis_col(d0_ref[...], d1_ref[...], blk)
        h2_ref[...] = lax.dot_general(
            x_ref[...] * dis, w_ref[...],
            (((1,), (1,)), ((), ())),
            preferred_element_type=jnp.float32,
        )

    grid = (deg0.shape[0] // r,)
    return pl.pallas_call(
        body,
        grid=grid,
        in_specs=[
            pl.BlockSpec((r, 128), lambda i: (i, 0)),
            pl.BlockSpec((r, 128), lambda i: (i, 0)),
            pl.BlockSpec((blk, d_in), lambda i: (i, 0)),
            pl.BlockSpec((d_out, d_in), lambda i: (0, 0)),
        ],
        out_specs=pl.BlockSpec((blk, d_out), lambda i: (i, 0)),
        out_shape=jax.ShapeDtypeStruct((n, d_out), jnp.float32),
    )(deg0, deg1, x, W)


def _epilogue(acc_partials, deg0, deg1, h2, b2, blk):
    nc, n_pad, d = acc_partials.shape
    n = h2.shape[0]
    r = blk // 128

    def body(ap_ref, d0_ref, d1_ref, h2_ref, b_ref, o_ref):
        dis = _dis_col(d0_ref[...], d1_ref[...], blk)
        s = ap_ref[0] + ap_ref[1] + h2_ref[...]
        o_ref[...] = s * dis + b_ref[...]

    return pl.pallas_call(
        body,
        grid=(n_pad // blk,),
        in_specs=[
            pl.BlockSpec((nc, blk, d), lambda i: (0, i, 0)),
            pl.BlockSpec((r, 128), lambda i: (i, 0)),
            pl.BlockSpec((r, 128), lambda i: (i, 0)),
            pl.BlockSpec((blk, d), lambda i: (i, 0)),
            pl.BlockSpec((1, d), lambda i: (0, 0)),
        ],
        out_specs=pl.BlockSpec((blk, d), lambda i: (i, 0)),
        out_shape=jax.ShapeDtypeStruct((n, d), jnp.float32),
    )(acc_partials, deg0, deg1, h2, b2)


def kernel(x, edge_index, edge_attr, W, b):
    n, d_in = x.shape
    d = W.shape[0]
    e = edge_index.shape[1]

    align = NS * 128
    n_pad = ((n + PAD_ROWS + align - 1) // align) * align
    cpt = -(-e // (NW * CH))
    cpt = ((cpt + 3) // 4) * 4
    e_pad = NW * cpt * CH
    stripe = n_pad // NS

    ei = edge_index.astype(jnp.int32)
    npad_e = e_pad - e
    pad_i = np.arange(npad_e, dtype=np.int32)
    pad2 = jnp.asarray(np.stack([pad_i % n, n + pad_i % (n_pad - n)]))
    ei_all = jnp.concatenate([ei, pad2], axis=1).reshape(2, NW, cpt, CH)

    zeros1 = jnp.zeros((stripe,), jnp.float32)
    zeros2 = jnp.zeros((stripe, d), jnp.float32)

    deg0, deg1 = _build_deg_kernel(n_pad, cpt)(ei_all, zeros1)
    deg0 = deg0.reshape(n_pad // 128, 128)
    deg1 = deg1.reshape(n_pad // 128, 128)
    h2 = _matmul_scale(deg0, deg1, x, W, blk=2048)
    acc_partials = _build_scatter_kernel(n_pad, cpt, d)(ei_all, h2, zeros2)
    return _epilogue(acc_partials, deg0, deg1, h2, b.reshape(1, d), blk=2048)
